# baseline (speedup 1.0000x reference)
import functools

import jax
import jax.numpy as jnp
from jax import lax
from jax.experimental import pallas as pl
from jax.experimental.pallas import tpu as pltpu
from jax.experimental.pallas import tpu_sc as plsc

NUM_CORES = 2
NUM_SUBCORES = 16
NUM_WORKERS = NUM_CORES * NUM_SUBCORES
CHUNK = 512

_mesh = plsc.VectorSubcoreMesh(core_axis_name="c", subcore_axis_name="s")


def _make_lookup(batch, dim):
    per_worker = batch // NUM_WORKERS
    n_chunks = per_worker // CHUNK
    n_pairs = n_chunks // 2

    @functools.partial(
        pl.kernel,
        mesh=_mesh,
        out_type=jax.ShapeDtypeStruct((batch, dim), jnp.float32),
        scratch_types=[
            pltpu.VMEM((per_worker,), jnp.int32),
            pltpu.VMEM((CHUNK, dim), jnp.float32),
            pltpu.VMEM((CHUNK, dim), jnp.float32),
            pltpu.SemaphoreType.DMA,
            pltpu.SemaphoreType.DMA,
            pltpu.SemaphoreType.DMA,
            pltpu.SemaphoreType.DMA,
        ],
        compiler_params=pltpu.CompilerParams(use_tc_tiling_on_sc=False),
    )
    def lookup(idx_hbm, table_hbm, out_hbm, idx_v, rows0, rows1,
               gsem0, gsem1, ssem0, ssem1):
        wid = lax.axis_index("s") * NUM_CORES + lax.axis_index("c")
        base = wid * per_worker
        pltpu.sync_copy(idx_hbm.at[pl.ds(base, per_worker)], idx_v)

        def fire(g, rows, gsem):
            pltpu.async_copy(
                table_hbm.at[idx_v.at[pl.ds(g * CHUNK, CHUNK)]], rows, gsem)

        def drain_gather(rows, gsem):
            pltpu.make_async_copy(
                table_hbm.at[pl.ds(0, CHUNK)], rows, gsem).wait()

        def start_store(g, rows, ssem):
            pltpu.async_copy(
                rows, out_hbm.at[pl.ds(base + g * CHUNK, CHUNK)], ssem)

        def wait_store(g, rows, ssem):
            pltpu.make_async_copy(
                rows, out_hbm.at[pl.ds(base + g * CHUNK, CHUNK)], ssem).wait()

        fire(0, rows0, gsem0)
        fire(1, rows1, gsem1)

        def body(p, carry):
            g0 = 2 * p
            g1 = 2 * p + 1
            drain_gather(rows0, gsem0)
            start_store(g0, rows0, ssem0)
            drain_gather(rows1, gsem1)
            start_store(g1, rows1, ssem1)

            @pl.when(p + 1 < n_pairs)
            def _():
                wait_store(g0, rows0, ssem0)
                fire(g0 + 2, rows0, gsem0)
                wait_store(g1, rows1, ssem1)
                fire(g1 + 2, rows1, gsem1)

            return carry

        lax.fori_loop(0, n_pairs, body, 0)
        wait_store(n_chunks - 2, rows0, ssem0)
        wait_store(n_chunks - 1, rows1, ssem1)

    return lookup


def kernel(token_ids, embedding_matrix):
    b, s = token_ids.shape
    _, dim = embedding_matrix.shape
    flat = token_ids.reshape(-1).astype(jnp.int32)
    out = _make_lookup(flat.shape[0], dim)(flat, embedding_matrix)
    return out.reshape(b, s, dim)


# speedup vs baseline: 1.0014x; 1.0014x over previous
import functools

import jax
import jax.numpy as jnp
from jax import lax
from jax.experimental import pallas as pl
from jax.experimental.pallas import tpu as pltpu
from jax.experimental.pallas import tpu_sc as plsc

NUM_CORES = 2
NUM_SUBCORES = 16
NUM_WORKERS = NUM_CORES * NUM_SUBCORES
CHUNK = 800

_mesh = plsc.VectorSubcoreMesh(core_axis_name="c", subcore_axis_name="s")


def _make_lookup(batch, dim):
    per_worker = batch // NUM_WORKERS
    n_chunks = per_worker // CHUNK
    n_pairs = n_chunks // 2

    @functools.partial(
        pl.kernel,
        mesh=_mesh,
        out_type=jax.ShapeDtypeStruct((batch, dim), jnp.float32),
        scratch_types=[
            pltpu.VMEM((per_worker,), jnp.int32),
            pltpu.VMEM((CHUNK, dim), jnp.float32),
            pltpu.VMEM((CHUNK, dim), jnp.float32),
            pltpu.SemaphoreType.DMA,
            pltpu.SemaphoreType.DMA,
            pltpu.SemaphoreType.DMA,
            pltpu.SemaphoreType.DMA,
        ],
        compiler_params=pltpu.CompilerParams(use_tc_tiling_on_sc=False),
    )
    def lookup(idx_hbm, table_hbm, out_hbm, idx_v, rows0, rows1,
               gsem0, gsem1, ssem0, ssem1):
        wid = lax.axis_index("s") * NUM_CORES + lax.axis_index("c")
        base = wid * per_worker
        pltpu.sync_copy(idx_hbm.at[pl.ds(base, per_worker)], idx_v)

        def fire(g, rows, gsem):
            pltpu.async_copy(
                table_hbm.at[idx_v.at[pl.ds(g * CHUNK, CHUNK)]], rows, gsem)

        def drain_gather(rows, gsem):
            pltpu.make_async_copy(
                table_hbm.at[pl.ds(0, CHUNK)], rows, gsem).wait()

        def start_store(g, rows, ssem):
            pltpu.async_copy(
                rows, out_hbm.at[pl.ds(base + g * CHUNK, CHUNK)], ssem)

        def wait_store(g, rows, ssem):
            pltpu.make_async_copy(
                rows, out_hbm.at[pl.ds(base + g * CHUNK, CHUNK)], ssem).wait()

        fire(0, rows0, gsem0)
        fire(1, rows1, gsem1)

        def body(p, carry):
            g0 = 2 * p
            g1 = 2 * p + 1
            drain_gather(rows0, gsem0)
            start_store(g0, rows0, ssem0)
            drain_gather(rows1, gsem1)
            start_store(g1, rows1, ssem1)

            @pl.when(p + 1 < n_pairs)
            def _():
                wait_store(g0, rows0, ssem0)
                fire(g0 + 2, rows0, gsem0)
                wait_store(g1, rows1, ssem1)
                fire(g1 + 2, rows1, gsem1)

            return carry

        lax.fori_loop(0, n_pairs, body, 0)
        wait_store(n_chunks - 2, rows0, ssem0)
        wait_store(n_chunks - 1, rows1, ssem1)

    return lookup


def kernel(token_ids, embedding_matrix):
    b, s = token_ids.shape
    _, dim = embedding_matrix.shape
    flat = token_ids.reshape(-1).astype(jnp.int32)
    out = _make_lookup(flat.shape[0], dim)(flat, embedding_matrix)
    return out.reshape(b, s, dim)


# speedup vs baseline: 1.0035x; 1.0021x over previous
import functools

import jax
import jax.numpy as jnp
from jax import lax
from jax.experimental import pallas as pl
from jax.experimental.pallas import tpu as pltpu
from jax.experimental.pallas import tpu_sc as plsc

NUM_CORES = 2
NUM_SUBCORES = 16
NUM_WORKERS = NUM_CORES * NUM_SUBCORES
CHUNK = 512

_mesh = plsc.VectorSubcoreMesh(core_axis_name="c", subcore_axis_name="s")


def _make_lookup(batch, dim):
    per_worker = batch // NUM_WORKERS
    n_chunks = per_worker // CHUNK
    n_pairs = n_chunks // 2

    @functools.partial(
        pl.kernel,
        mesh=_mesh,
        out_type=jax.ShapeDtypeStruct((batch, dim), jnp.float32),
        scratch_types=[
            pltpu.VMEM((per_worker,), jnp.int32),
            pltpu.VMEM((CHUNK, dim), jnp.float32),
            pltpu.VMEM((CHUNK, dim), jnp.float32),
            pltpu.SemaphoreType.DMA,
            pltpu.SemaphoreType.DMA,
            pltpu.SemaphoreType.DMA,
            pltpu.SemaphoreType.DMA,
        ],
        compiler_params=pltpu.CompilerParams(use_tc_tiling_on_sc=False),
    )
    def lookup(idx_hbm, table_hbm, out_hbm, idx_v, rows0, rows1,
               gsem0, gsem1, ssem0, ssem1):
        wid = lax.axis_index("s") * NUM_CORES + lax.axis_index("c")
        base = wid * per_worker
        pltpu.sync_copy(idx_hbm.at[pl.ds(base, per_worker)], idx_v)

        def fire(g, rows, gsem):
            pltpu.async_copy(
                table_hbm.at[idx_v.at[pl.ds(g * CHUNK, CHUNK)]], rows, gsem)

        def drain_gather(rows, gsem):
            pltpu.make_async_copy(
                table_hbm.at[pl.ds(0, CHUNK)], rows, gsem).wait()

        def start_store(g, rows, ssem):
            pltpu.async_copy(
                rows, out_hbm.at[pl.ds(base + g * CHUNK, CHUNK)], ssem)

        def wait_store(g, rows, ssem):
            pltpu.make_async_copy(
                rows, out_hbm.at[pl.ds(base + g * CHUNK, CHUNK)], ssem).wait()

        fire(0, rows0, gsem0)
        fire(1, rows1, gsem1)

        def body(p, carry):
            g0 = 2 * p
            g1 = 2 * p + 1
            drain_gather(rows0, gsem0)
            start_store(g0, rows0, ssem0)
            drain_gather(rows1, gsem1)
            start_store(g1, rows1, ssem1)

            @pl.when(p + 1 < n_pairs)
            def _():
                wait_store(g0, rows0, ssem0)
                fire(g0 + 2, rows0, gsem0)
                wait_store(g1, rows1, ssem1)
                fire(g1 + 2, rows1, gsem1)

            return carry

        lax.fori_loop(0, n_pairs, body, 0)
        wait_store(n_chunks - 2, rows0, ssem0)
        wait_store(n_chunks - 1, rows1, ssem1)

    return lookup


def kernel(token_ids, embedding_matrix):
    b, s = token_ids.shape
    _, dim = embedding_matrix.shape
    flat = token_ids.reshape(-1).astype(jnp.int32)
    out = _make_lookup(flat.shape[0], dim)(flat, embedding_matrix)
    return out.reshape(b, s, dim)


# speedup vs baseline: 1.0046x; 1.0011x over previous
import functools

import jax
import jax.numpy as jnp
from jax import lax
from jax.experimental import pallas as pl
from jax.experimental.pallas import tpu as pltpu
from jax.experimental.pallas import tpu_sc as plsc

NUM_CORES = 2
NUM_SUBCORES = 16
NUM_WORKERS = NUM_CORES * NUM_SUBCORES
CHUNK = 256
NBUF = 4

_mesh = plsc.VectorSubcoreMesh(core_axis_name="c", subcore_axis_name="s")


def _make_lookup(batch, dim):
    per_worker = batch // NUM_WORKERS
    n_chunks = per_worker // CHUNK
    n_groups = n_chunks // NBUF

    @functools.partial(
        pl.kernel,
        mesh=_mesh,
        out_type=jax.ShapeDtypeStruct((batch, dim), jnp.float32),
        scratch_types=(
            [pltpu.VMEM((per_worker,), jnp.int32)]
            + [pltpu.VMEM((CHUNK, dim), jnp.float32)] * NBUF
            + [pltpu.SemaphoreType.DMA] * (2 * NBUF)
        ),
        compiler_params=pltpu.CompilerParams(use_tc_tiling_on_sc=False),
    )
    def lookup(idx_hbm, table_hbm, out_hbm, idx_v, *bufs_and_sems):
        rows = bufs_and_sems[:NBUF]
        gsems = bufs_and_sems[NBUF:2 * NBUF]
        ssems = bufs_and_sems[2 * NBUF:]
        wid = lax.axis_index("s") * NUM_CORES + lax.axis_index("c")
        base = wid * per_worker
        pltpu.sync_copy(idx_hbm.at[pl.ds(base, per_worker)], idx_v)

        def fire(g, i):
            pltpu.async_copy(
                table_hbm.at[idx_v.at[pl.ds(g * CHUNK, CHUNK)]],
                rows[i], gsems[i])

        def drain_gather(i):
            pltpu.make_async_copy(
                table_hbm.at[pl.ds(0, CHUNK)], rows[i], gsems[i]).wait()

        def start_store(g, i):
            pltpu.async_copy(
                rows[i], out_hbm.at[pl.ds(base + g * CHUNK, CHUNK)], ssems[i])

        def wait_store(g, i):
            pltpu.make_async_copy(
                rows[i], out_hbm.at[pl.ds(base + g * CHUNK, CHUNK)],
                ssems[i]).wait()

        for i in range(NBUF):
            fire(i, i)

        def body(q, carry):
            g = q * NBUF
            for i in range(NBUF):
                drain_gather(i)
                start_store(g + i, i)

            @pl.when(q + 1 < n_groups)
            def _():
                for i in range(NBUF):
                    wait_store(g + i, i)
                    fire(g + i + NBUF, i)

            return carry

        lax.fori_loop(0, n_groups, body, 0)
        for i in range(NBUF):
            wait_store(n_chunks - NBUF + i, i)

    return lookup


def kernel(token_ids, embedding_matrix):
    b, s = token_ids.shape
    _, dim = embedding_matrix.shape
    flat = token_ids.reshape(-1).astype(jnp.int32)
    out = _make_lookup(flat.shape[0], dim)(flat, embedding_matrix)
    return out.reshape(b, s, dim)
